# baseline (speedup 1.0000x reference)
import functools

import jax
import jax.numpy as jnp
from jax import lax
from jax.experimental import pallas as pl
from jax.experimental.pallas import tpu as pltpu
from jax.experimental.pallas import tpu_sc as plsc

_N = 4096
_D = _N + 1
_EPS = 1e-6
_K = 4
_RPW = 128
_NCHUNK = _RPW // _K
_VPAD = _N + 16


def _spu(x):
    return jnp.where(x >= 0, x * x - 0.5, jax.nn.sigmoid(-x) - 1.0)


def _spu_grad(x):
    s = jax.nn.sigmoid(-x)
    return jnp.where(x >= 0, 2.0 * x, -s * (1.0 - s))


def _diff_clamp(x, a, b):
    return jnp.tanh(x) * (b - a) / 2.0 + (b + a) / 2.0


def _params_body(lb_ref, ub_ref, sl_ref, su_ref,
                 slu_ref, suu_ref, lbias_ref, ubias_ref):
    lb = lb_ref[...]
    ub = ub_ref[...]
    slope_l = sl_ref[...]
    slope_u = su_ref[...]
    spu_ub = _spu(ub)
    spu_lb = _spu(lb)
    g_ub = _spu_grad(ub)
    g_lb = _spu_grad(lb)
    mask_1 = lb >= 0
    mask_2 = ub <= 0
    a = (spu_ub - spu_lb) / (ub - lb + _EPS)
    zeros = jnp.zeros_like(a)
    slope_u_use = jnp.where(
        mask_1,
        _diff_clamp(slope_u, a, a),
        jnp.where(
            mask_2,
            _diff_clamp(slope_u, g_ub, g_lb),
            _diff_clamp(slope_u, jnp.full_like(a, -0.25), jnp.maximum(zeros, a)),
        ),
    )
    slope_l_use = jnp.where(
        mask_1,
        _diff_clamp(slope_l, g_lb, g_ub),
        jnp.where(
            mask_2,
            _diff_clamp(slope_l, a, a),
            _diff_clamp(slope_l, (spu_lb + 0.5) / (lb + _EPS), g_ub),
        ),
    )
    b1 = spu_lb - slope_l_use * lb
    b2 = spu_ub - slope_l_use * ub
    l_bias = jnp.minimum(b1, b2)
    c1 = spu_lb - slope_u_use * lb
    c2 = spu_ub - slope_u_use * ub
    xv = slope_u_use / 2.0
    valid = (xv >= jnp.maximum(lb, 0.0)) & (xv <= ub)
    c3 = jnp.where(valid, -slope_u_use * slope_u_use / 4.0 - 0.5, -1e30)
    u_bias = jnp.maximum(jnp.maximum(c1, c2), c3)
    slu_ref[...] = slope_l_use
    suu_ref[...] = slope_u_use
    lbias_ref[...] = l_bias
    ubias_ref[...] = u_bias


def _compute_params(lb, ub, slope_l, slope_u):
    shape2d = (32, 128)
    args = [x.reshape(shape2d) for x in (lb, ub, slope_l, slope_u)]
    o = jax.ShapeDtypeStruct(shape2d, jnp.float32)
    slu, suu, lbias, ubias = pl.pallas_call(
        _params_body,
        out_shape=[o, o, o, o],
    )(*args)
    return (slu.reshape(_N), suu.reshape(_N),
            lbias.reshape(_N), ubias.reshape(_N))


def _sc_fill_body(slu_hbm, suu_hbm, lbias_hbm, ubias_hbm, zrows_hbm,
                  out_hbm, slu_v, suu_v, lb_v, ub_v,
                  buf0, buf1, tr_buf, sem0, sem1, tr_sem):
    c = lax.axis_index("c")
    s = lax.axis_index("s")
    w = c * 16 + s

    pltpu.sync_copy(slu_hbm, slu_v.at[pl.ds(0, _N)])
    pltpu.sync_copy(suu_hbm, suu_v.at[pl.ds(0, _N)])
    pltpu.sync_copy(lbias_hbm, lb_v.at[pl.ds(0, _N)])
    pltpu.sync_copy(ubias_hbm, ub_v.at[pl.ds(0, _N)])

    pltpu.sync_copy(zrows_hbm, buf0)
    pltpu.sync_copy(zrows_hbm, buf1)

    bufs = (buf0, buf1)
    sems = (sem0, sem1)

    lane = lax.iota(jnp.int32, 16)
    rr = lane & 3
    mlow = lane < 4
    zero16i = jnp.zeros((16,), jnp.int32)
    one16i = jnp.full((16,), 1, jnp.int32)
    last16i = jnp.full((16,), _N, jnp.int32)
    zval = jnp.zeros((16,), jnp.float32)
    row0 = w * _RPW

    def scatter(b, r0, vals):
        dcol = r0 + rr
        if vals is None:
            v = (zval, zval, zval, zval)
        else:
            v = vals
        plsc.store_scatter(bufs[b], [rr, zero16i, dcol], v[0], mask=mlow)
        plsc.store_scatter(bufs[b], [rr, one16i, dcol], v[1], mask=mlow)
        plsc.store_scatter(bufs[b], [rr, zero16i, last16i], v[2], mask=mlow)
        plsc.store_scatter(bufs[b], [rr, one16i, last16i], v[3], mask=mlow)

    def load_vals(r0):
        return (slu_v[pl.ds(r0, 16)], suu_v[pl.ds(r0, 16)],
                lb_v[pl.ds(r0, 16)], ub_v[pl.ds(r0, 16)])

    def start_dma(b, r0):
        pltpu.async_copy(bufs[b], out_hbm.at[pl.ds(r0, _K)], sems[b])

    def wait_dma(b):
        pltpu.make_async_copy(bufs[b], out_hbm.at[pl.ds(0, _K)], sems[b]).wait()

    for b in range(2):
        r0 = row0 + b * _K
        scatter(b, r0, load_vals(r0))
        start_dma(b, r0)

    def pair(j, carry):
        for b in range(2):
            ii = 2 * j + b
            r0 = row0 + ii * _K
            wait_dma(b)
            scatter(b, r0 - 2 * _K, None)
            scatter(b, r0, load_vals(r0))
            start_dma(b, r0)
        return carry

    lax.fori_loop(1, _NCHUNK // 2, pair, 0)
    wait_dma(0)
    wait_dma(1)

    @pl.when(w == 31)
    def _():
        pltpu.sync_copy(zrows_hbm.at[pl.ds(0, 1)], tr_buf)
        m_idx = jnp.where(lane < 8, 0, 1).astype(jnp.int32)
        one_val = jnp.full((16,), 1.0, jnp.float32)
        plsc.store_scatter(tr_buf, [zero16i, m_idx, last16i], one_val)
        pltpu.async_copy(tr_buf, out_hbm.at[pl.ds(_N, 1)], tr_sem).wait()


def _sc_fill(slu, suu, lbias, ubias):
    zrows = jnp.zeros((_K, 2, _D), jnp.float32)
    mesh = plsc.VectorSubcoreMesh(core_axis_name="c", subcore_axis_name="s")
    fill = functools.partial(
        pl.kernel,
        mesh=mesh,
        compiler_params=pltpu.CompilerParams(needs_layout_passes=False),
        out_type=jax.ShapeDtypeStruct((_D, 2, _D), jnp.float32),
        scratch_types=[
            pltpu.VMEM((_VPAD,), jnp.float32),
            pltpu.VMEM((_VPAD,), jnp.float32),
            pltpu.VMEM((_VPAD,), jnp.float32),
            pltpu.VMEM((_VPAD,), jnp.float32),
            pltpu.VMEM((_K, 2, _D), jnp.float32),
            pltpu.VMEM((_K, 2, _D), jnp.float32),
            pltpu.VMEM((1, 2, _D), jnp.float32),
            pltpu.SemaphoreType.DMA,
            pltpu.SemaphoreType.DMA,
            pltpu.SemaphoreType.DMA,
        ],
    )(_sc_fill_body)
    return fill(slu, suu, lbias, ubias, zrows)


def kernel(lb, ub, slope_l, slope_u):
    slu, suu, lbias, ubias = _compute_params(lb, ub, slope_l, slope_u)
    out = _sc_fill(slu, suu, lbias, ubias)
    return jnp.transpose(out, (1, 0, 2))


# speedup vs baseline: 1.2357x; 1.2357x over previous
import functools

import jax
import jax.numpy as jnp
from jax import lax
from jax.experimental import pallas as pl
from jax.experimental.pallas import tpu as pltpu
from jax.experimental.pallas import tpu_sc as plsc

_N = 4096
_D = _N + 1
_EPS = 1e-6
_K = 8
_RPW = 128
_NCHUNK = _RPW // _K
_VSTAGE = 256


def _spu(x):
    return jnp.where(x >= 0, x * x - 0.5, jax.nn.sigmoid(-x) - 1.0)


def _spu_grad(x):
    s = jax.nn.sigmoid(-x)
    return jnp.where(x >= 0, 2.0 * x, -s * (1.0 - s))


def _diff_clamp(x, a, b):
    return jnp.tanh(x) * (b - a) / 2.0 + (b + a) / 2.0


def _params_body(lb_ref, ub_ref, sl_ref, su_ref,
                 slu_ref, suu_ref, lbias_ref, ubias_ref):
    lb = lb_ref[...]
    ub = ub_ref[...]
    slope_l = sl_ref[...]
    slope_u = su_ref[...]
    spu_ub = _spu(ub)
    spu_lb = _spu(lb)
    g_ub = _spu_grad(ub)
    g_lb = _spu_grad(lb)
    mask_1 = lb >= 0
    mask_2 = ub <= 0
    a = (spu_ub - spu_lb) / (ub - lb + _EPS)
    zeros = jnp.zeros_like(a)
    slope_u_use = jnp.where(
        mask_1,
        _diff_clamp(slope_u, a, a),
        jnp.where(
            mask_2,
            _diff_clamp(slope_u, g_ub, g_lb),
            _diff_clamp(slope_u, jnp.full_like(a, -0.25), jnp.maximum(zeros, a)),
        ),
    )
    slope_l_use = jnp.where(
        mask_1,
        _diff_clamp(slope_l, g_lb, g_ub),
        jnp.where(
            mask_2,
            _diff_clamp(slope_l, a, a),
            _diff_clamp(slope_l, (spu_lb + 0.5) / (lb + _EPS), g_ub),
        ),
    )
    b1 = spu_lb - slope_l_use * lb
    b2 = spu_ub - slope_l_use * ub
    l_bias = jnp.minimum(b1, b2)
    c1 = spu_lb - slope_u_use * lb
    c2 = spu_ub - slope_u_use * ub
    xv = slope_u_use / 2.0
    valid = (xv >= jnp.maximum(lb, 0.0)) & (xv <= ub)
    c3 = jnp.where(valid, -slope_u_use * slope_u_use / 4.0 - 0.5, -1e30)
    u_bias = jnp.maximum(jnp.maximum(c1, c2), c3)
    slu_ref[...] = slope_l_use
    suu_ref[...] = slope_u_use
    lbias_ref[...] = l_bias
    ubias_ref[...] = u_bias


def _compute_params(lb, ub, slope_l, slope_u):
    shape2d = (32, 128)
    args = [x.reshape(shape2d) for x in (lb, ub, slope_l, slope_u)]
    o = jax.ShapeDtypeStruct(shape2d, jnp.float32)
    slu, suu, lbias, ubias = pl.pallas_call(
        _params_body,
        out_shape=[o, o, o, o],
    )(*args)
    return jnp.stack(
        [slu.reshape(_N), suu.reshape(_N),
         lbias.reshape(_N), ubias.reshape(_N)])


def _sc_fill_body(params_hbm, zrows_hbm,
                  out_hbm, pv, buf, sem):
    c = lax.axis_index("c")
    s = lax.axis_index("s")
    w = c * 16 + s
    row0 = w * _RPW

    pltpu.sync_copy(params_hbm.at[:, pl.ds(row0, _VSTAGE)], pv)

    pltpu.sync_copy(zrows_hbm, buf)

    lane = lax.iota(jnp.int32, 16)
    rr = lane & 7
    mlow = lane < 8
    zero16i = jnp.zeros((16,), jnp.int32)
    one16i = jnp.full((16,), 1, jnp.int32)
    last16i = jnp.full((16,), _N, jnp.int32)
    zval = jnp.zeros((16,), jnp.float32)

    def chunk(i, carry):
        l0 = i * _K
        r0 = row0 + l0
        dcol = r0 + rr
        slu16 = pv[0, pl.ds(l0, 16)]
        suu16 = pv[1, pl.ds(l0, 16)]
        lb16 = pv[2, pl.ds(l0, 16)]
        ub16 = pv[3, pl.ds(l0, 16)]
        plsc.store_scatter(buf, [rr, zero16i, dcol], slu16, mask=mlow)
        plsc.store_scatter(buf, [rr, one16i, dcol], suu16, mask=mlow)
        plsc.store_scatter(buf, [rr, zero16i, last16i], lb16, mask=mlow)
        plsc.store_scatter(buf, [rr, one16i, last16i], ub16, mask=mlow)
        pltpu.async_copy(buf, out_hbm.at[pl.ds(r0, _K)], sem).wait()
        plsc.store_scatter(buf, [rr, zero16i, dcol], zval, mask=mlow)
        plsc.store_scatter(buf, [rr, one16i, dcol], zval, mask=mlow)
        plsc.store_scatter(buf, [rr, zero16i, last16i], zval, mask=mlow)
        plsc.store_scatter(buf, [rr, one16i, last16i], zval, mask=mlow)
        return carry

    lax.fori_loop(0, _NCHUNK, chunk, 0)

    @pl.when(w == 31)
    def _():
        m_idx = jnp.where(lane < 8, 0, 1).astype(jnp.int32)
        one_val = jnp.full((16,), 1.0, jnp.float32)
        plsc.store_scatter(buf, [zero16i, m_idx, last16i], one_val)
        pltpu.async_copy(buf.at[pl.ds(0, 1)],
                         out_hbm.at[pl.ds(_N, 1)], sem).wait()


def _sc_fill(params):
    params = jnp.concatenate(
        [params, jnp.zeros((4, _VSTAGE - _RPW), jnp.float32)], axis=1)
    zrows = jnp.zeros((_K, 2, _D), jnp.float32)
    mesh = plsc.VectorSubcoreMesh(core_axis_name="c", subcore_axis_name="s")
    fill = functools.partial(
        pl.kernel,
        mesh=mesh,
        compiler_params=pltpu.CompilerParams(needs_layout_passes=False),
        out_type=jax.ShapeDtypeStruct((_D, 2, _D), jnp.float32),
        scratch_types=[
            pltpu.VMEM((4, _VSTAGE), jnp.float32),
            pltpu.VMEM((_K, 2, _D), jnp.float32),
            pltpu.SemaphoreType.DMA,
        ],
    )(_sc_fill_body)
    return fill(params, zrows)


def kernel(lb, ub, slope_l, slope_u):
    params = _compute_params(lb, ub, slope_l, slope_u)
    out = _sc_fill(params)
    return jnp.transpose(out, (1, 0, 2))


# speedup vs baseline: 1.2385x; 1.0023x over previous
import functools

import jax
import jax.numpy as jnp
from jax import lax
from jax.experimental import pallas as pl
from jax.experimental.pallas import tpu as pltpu
from jax.experimental.pallas import tpu_sc as plsc

_N = 4096
_D = _N + 1
_EPS = 1e-6
_K = 8
_RPW = 128
_NCHUNK = _RPW // _K
_VSTAGE = 256


def _spu(x):
    return jnp.where(x >= 0, x * x - 0.5, jax.nn.sigmoid(-x) - 1.0)


def _spu_grad(x):
    s = jax.nn.sigmoid(-x)
    return jnp.where(x >= 0, 2.0 * x, -s * (1.0 - s))


def _diff_clamp(x, a, b):
    return jnp.tanh(x) * (b - a) / 2.0 + (b + a) / 2.0


def _params_body(lb_ref, ub_ref, sl_ref, su_ref,
                 slu_ref, suu_ref, lbias_ref, ubias_ref):
    lb = lb_ref[...]
    ub = ub_ref[...]
    slope_l = sl_ref[...]
    slope_u = su_ref[...]
    spu_ub = _spu(ub)
    spu_lb = _spu(lb)
    g_ub = _spu_grad(ub)
    g_lb = _spu_grad(lb)
    mask_1 = lb >= 0
    mask_2 = ub <= 0
    a = (spu_ub - spu_lb) / (ub - lb + _EPS)
    zeros = jnp.zeros_like(a)
    slope_u_use = jnp.where(
        mask_1,
        _diff_clamp(slope_u, a, a),
        jnp.where(
            mask_2,
            _diff_clamp(slope_u, g_ub, g_lb),
            _diff_clamp(slope_u, jnp.full_like(a, -0.25), jnp.maximum(zeros, a)),
        ),
    )
    slope_l_use = jnp.where(
        mask_1,
        _diff_clamp(slope_l, g_lb, g_ub),
        jnp.where(
            mask_2,
            _diff_clamp(slope_l, a, a),
            _diff_clamp(slope_l, (spu_lb + 0.5) / (lb + _EPS), g_ub),
        ),
    )
    b1 = spu_lb - slope_l_use * lb
    b2 = spu_ub - slope_l_use * ub
    l_bias = jnp.minimum(b1, b2)
    c1 = spu_lb - slope_u_use * lb
    c2 = spu_ub - slope_u_use * ub
    xv = slope_u_use / 2.0
    valid = (xv >= jnp.maximum(lb, 0.0)) & (xv <= ub)
    c3 = jnp.where(valid, -slope_u_use * slope_u_use / 4.0 - 0.5, -1e30)
    u_bias = jnp.maximum(jnp.maximum(c1, c2), c3)
    slu_ref[...] = slope_l_use
    suu_ref[...] = slope_u_use
    lbias_ref[...] = l_bias
    ubias_ref[...] = u_bias


def _compute_params(lb, ub, slope_l, slope_u):
    shape2d = (32, 128)
    args = [x.reshape(shape2d) for x in (lb, ub, slope_l, slope_u)]
    o = jax.ShapeDtypeStruct(shape2d, jnp.float32)
    slu, suu, lbias, ubias = pl.pallas_call(
        _params_body,
        out_shape=[o, o, o, o],
    )(*args)
    return jnp.stack(
        [slu.reshape(_N), suu.reshape(_N),
         lbias.reshape(_N), ubias.reshape(_N)])


def _sc_fill_body(params_hbm, zrows_hbm,
                  out_hbm, pv, buf, sem):
    c = lax.axis_index("c")
    s = lax.axis_index("s")
    w = s * 2 + c
    row0 = w * _RPW

    pltpu.sync_copy(params_hbm.at[:, pl.ds(row0, _VSTAGE)], pv)

    pltpu.sync_copy(zrows_hbm, buf)

    lane = lax.iota(jnp.int32, 16)
    rr = lane & 7
    mlow = lane < 8
    zero16i = jnp.zeros((16,), jnp.int32)
    one16i = jnp.full((16,), 1, jnp.int32)
    last16i = jnp.full((16,), _N, jnp.int32)
    zval = jnp.zeros((16,), jnp.float32)

    def chunk(i, carry):
        l0 = i * _K
        r0 = row0 + l0
        dcol = r0 + rr
        slu16 = pv[0, pl.ds(l0, 16)]
        suu16 = pv[1, pl.ds(l0, 16)]
        lb16 = pv[2, pl.ds(l0, 16)]
        ub16 = pv[3, pl.ds(l0, 16)]
        plsc.store_scatter(buf, [rr, zero16i, dcol], slu16, mask=mlow)
        plsc.store_scatter(buf, [rr, one16i, dcol], suu16, mask=mlow)
        plsc.store_scatter(buf, [rr, zero16i, last16i], lb16, mask=mlow)
        plsc.store_scatter(buf, [rr, one16i, last16i], ub16, mask=mlow)
        pltpu.async_copy(buf, out_hbm.at[pl.ds(r0, _K)], sem).wait()
        plsc.store_scatter(buf, [rr, zero16i, dcol], zval, mask=mlow)
        plsc.store_scatter(buf, [rr, one16i, dcol], zval, mask=mlow)
        plsc.store_scatter(buf, [rr, zero16i, last16i], zval, mask=mlow)
        plsc.store_scatter(buf, [rr, one16i, last16i], zval, mask=mlow)
        return carry

    lax.fori_loop(0, _NCHUNK, chunk, 0)

    @pl.when(w == 31)
    def _():
        m_idx = jnp.where(lane < 8, 0, 1).astype(jnp.int32)
        one_val = jnp.full((16,), 1.0, jnp.float32)
        plsc.store_scatter(buf, [zero16i, m_idx, last16i], one_val)
        pltpu.async_copy(buf.at[pl.ds(0, 1)],
                         out_hbm.at[pl.ds(_N, 1)], sem).wait()


def _sc_fill(params):
    params = jnp.concatenate(
        [params, jnp.zeros((4, _VSTAGE - _RPW), jnp.float32)], axis=1)
    zrows = jnp.zeros((_K, 2, _D), jnp.float32)
    mesh = plsc.VectorSubcoreMesh(core_axis_name="c", subcore_axis_name="s")
    fill = functools.partial(
        pl.kernel,
        mesh=mesh,
        compiler_params=pltpu.CompilerParams(needs_layout_passes=False),
        out_type=jax.ShapeDtypeStruct((_D, 2, _D), jnp.float32),
        scratch_types=[
            pltpu.VMEM((4, _VSTAGE), jnp.float32),
            pltpu.VMEM((_K, 2, _D), jnp.float32),
            pltpu.SemaphoreType.DMA,
        ],
    )(_sc_fill_body)
    return fill(params, zrows)


def kernel(lb, ub, slope_l, slope_u):
    params = _compute_params(lb, ub, slope_l, slope_u)
    out = _sc_fill(params)
    return jnp.transpose(out, (1, 0, 2))


# speedup vs baseline: 1.2410x; 1.0020x over previous
import functools

import jax
import jax.numpy as jnp
from jax import lax
from jax.experimental import pallas as pl
from jax.experimental.pallas import tpu as pltpu
from jax.experimental.pallas import tpu_sc as plsc

_N = 4096
_D = _N + 1
_EPS = 1e-6
_K = 8
_RPW = 128
_NCHUNK = _RPW // _K
_VSTAGE = 256
_PW = _N + _VSTAGE - _RPW


def _spu(x):
    return jnp.where(x >= 0, x * x - 0.5, jax.nn.sigmoid(-x) - 1.0)


def _spu_grad(x):
    s = jax.nn.sigmoid(-x)
    return jnp.where(x >= 0, 2.0 * x, -s * (1.0 - s))


def _diff_clamp(x, a, b):
    return jnp.tanh(x) * (b - a) / 2.0 + (b + a) / 2.0


def _params_body(lb_ref, ub_ref, sl_ref, su_ref, params_ref):
    lb = lb_ref[...]
    ub = ub_ref[...]
    slope_l = sl_ref[...]
    slope_u = su_ref[...]
    spu_ub = _spu(ub)
    spu_lb = _spu(lb)
    g_ub = _spu_grad(ub)
    g_lb = _spu_grad(lb)
    mask_1 = lb >= 0
    mask_2 = ub <= 0
    a = (spu_ub - spu_lb) / (ub - lb + _EPS)
    zeros = jnp.zeros_like(a)
    slope_u_use = jnp.where(
        mask_1,
        _diff_clamp(slope_u, a, a),
        jnp.where(
            mask_2,
            _diff_clamp(slope_u, g_ub, g_lb),
            _diff_clamp(slope_u, jnp.full_like(a, -0.25), jnp.maximum(zeros, a)),
        ),
    )
    slope_l_use = jnp.where(
        mask_1,
        _diff_clamp(slope_l, g_lb, g_ub),
        jnp.where(
            mask_2,
            _diff_clamp(slope_l, a, a),
            _diff_clamp(slope_l, (spu_lb + 0.5) / (lb + _EPS), g_ub),
        ),
    )
    b1 = spu_lb - slope_l_use * lb
    b2 = spu_ub - slope_l_use * ub
    l_bias = jnp.minimum(b1, b2)
    c1 = spu_lb - slope_u_use * lb
    c2 = spu_ub - slope_u_use * ub
    xv = slope_u_use / 2.0
    valid = (xv >= jnp.maximum(lb, 0.0)) & (xv <= ub)
    c3 = jnp.where(valid, -slope_u_use * slope_u_use / 4.0 - 0.5, -1e30)
    u_bias = jnp.maximum(jnp.maximum(c1, c2), c3)
    params_ref[0:1, 0:_N] = slope_l_use
    params_ref[1:2, 0:_N] = slope_u_use
    params_ref[2:3, 0:_N] = l_bias
    params_ref[3:4, 0:_N] = u_bias
    params_ref[:, _N:] = jnp.zeros((4, _PW - _N), jnp.float32)


def _compute_params(lb, ub, slope_l, slope_u):
    args = [x.reshape(1, _N) for x in (lb, ub, slope_l, slope_u)]
    return pl.pallas_call(
        _params_body,
        out_shape=jax.ShapeDtypeStruct((4, _PW), jnp.float32),
    )(*args)


def _sc_fill_body(params_hbm, zrows_hbm,
                  out_hbm, pv, buf, sem):
    c = lax.axis_index("c")
    s = lax.axis_index("s")
    w = s * 2 + c
    row0 = w * _RPW

    pltpu.sync_copy(params_hbm.at[:, pl.ds(row0, _VSTAGE)], pv)

    pltpu.sync_copy(zrows_hbm, buf)

    lane = lax.iota(jnp.int32, 16)
    rr = lane & 7
    mlow = lane < 8
    zero16i = jnp.zeros((16,), jnp.int32)
    one16i = jnp.full((16,), 1, jnp.int32)
    last16i = jnp.full((16,), _N, jnp.int32)
    zval = jnp.zeros((16,), jnp.float32)

    def chunk(i, carry):
        l0 = i * _K
        r0 = row0 + l0
        dcol = r0 + rr
        slu16 = pv[0, pl.ds(l0, 16)]
        suu16 = pv[1, pl.ds(l0, 16)]
        lb16 = pv[2, pl.ds(l0, 16)]
        ub16 = pv[3, pl.ds(l0, 16)]
        plsc.store_scatter(buf, [rr, zero16i, dcol], slu16, mask=mlow)
        plsc.store_scatter(buf, [rr, one16i, dcol], suu16, mask=mlow)
        plsc.store_scatter(buf, [rr, zero16i, last16i], lb16, mask=mlow)
        plsc.store_scatter(buf, [rr, one16i, last16i], ub16, mask=mlow)
        pltpu.async_copy(buf, out_hbm.at[pl.ds(r0, _K)], sem).wait()
        plsc.store_scatter(buf, [rr, zero16i, dcol], zval, mask=mlow)
        plsc.store_scatter(buf, [rr, one16i, dcol], zval, mask=mlow)
        plsc.store_scatter(buf, [rr, zero16i, last16i], zval, mask=mlow)
        plsc.store_scatter(buf, [rr, one16i, last16i], zval, mask=mlow)
        return carry

    lax.fori_loop(0, _NCHUNK, chunk, 0)

    @pl.when(w == 31)
    def _():
        m_idx = jnp.where(lane < 8, 0, 1).astype(jnp.int32)
        one_val = jnp.full((16,), 1.0, jnp.float32)
        plsc.store_scatter(buf, [zero16i, m_idx, last16i], one_val)
        pltpu.async_copy(buf.at[pl.ds(0, 1)],
                         out_hbm.at[pl.ds(_N, 1)], sem).wait()


def _sc_fill(params):
    zrows = jnp.zeros((_K, 2, _D), jnp.float32)
    mesh = plsc.VectorSubcoreMesh(core_axis_name="c", subcore_axis_name="s")
    fill = functools.partial(
        pl.kernel,
        mesh=mesh,
        compiler_params=pltpu.CompilerParams(needs_layout_passes=False),
        out_type=jax.ShapeDtypeStruct((_D, 2, _D), jnp.float32),
        scratch_types=[
            pltpu.VMEM((4, _VSTAGE), jnp.float32),
            pltpu.VMEM((_K, 2, _D), jnp.float32),
            pltpu.SemaphoreType.DMA,
        ],
    )(_sc_fill_body)
    return fill(params, zrows)


def kernel(lb, ub, slope_l, slope_u):
    params = _compute_params(lb, ub, slope_l, slope_u)
    out = _sc_fill(params)
    return jnp.transpose(out, (1, 0, 2))


# speedup vs baseline: 1.2412x; 1.0002x over previous
import functools

import jax
import jax.numpy as jnp
from jax import lax
from jax.experimental import pallas as pl
from jax.experimental.pallas import tpu as pltpu
from jax.experimental.pallas import tpu_sc as plsc

_N = 4096
_D = _N + 1
_EPS = 1e-6
_K = 8
_RPW = 128
_NCHUNK = _RPW // _K
_PVW = 144


def _sigmoid(x):
    return 1.0 / (1.0 + jnp.exp(-x))


def _tanh(x):
    return 1.0 - 2.0 / (jnp.exp(2.0 * x) + 1.0)


def _spu(x):
    return jnp.where(x >= 0, x * x - 0.5, _sigmoid(-x) - 1.0)


def _spu_grad(x):
    s = _sigmoid(-x)
    return jnp.where(x >= 0, 2.0 * x, -s * (1.0 - s))


def _diff_clamp(x, a, b):
    return _tanh(x) * (b - a) / 2.0 + (b + a) / 2.0


def _params16(lb, ub, slope_l, slope_u):
    spu_ub = _spu(ub)
    spu_lb = _spu(lb)
    g_ub = _spu_grad(ub)
    g_lb = _spu_grad(lb)
    mask_1 = lb >= 0
    mask_2 = ub <= 0
    a = (spu_ub - spu_lb) / (ub - lb + _EPS)
    zeros = jnp.zeros_like(a)
    slope_u_use = jnp.where(
        mask_1,
        _diff_clamp(slope_u, a, a),
        jnp.where(
            mask_2,
            _diff_clamp(slope_u, g_ub, g_lb),
            _diff_clamp(slope_u, jnp.full_like(a, -0.25), jnp.maximum(zeros, a)),
        ),
    )
    slope_l_use = jnp.where(
        mask_1,
        _diff_clamp(slope_l, g_lb, g_ub),
        jnp.where(
            mask_2,
            _diff_clamp(slope_l, a, a),
            _diff_clamp(slope_l, (spu_lb + 0.5) / (lb + _EPS), g_ub),
        ),
    )
    b1 = spu_lb - slope_l_use * lb
    b2 = spu_ub - slope_l_use * ub
    l_bias = jnp.minimum(b1, b2)
    c1 = spu_lb - slope_u_use * lb
    c2 = spu_ub - slope_u_use * ub
    xv = slope_u_use / 2.0
    valid = (xv >= jnp.maximum(lb, 0.0)) & (xv <= ub)
    c3 = jnp.where(valid, -slope_u_use * slope_u_use / 4.0 - 0.5, -1e30)
    u_bias = jnp.maximum(jnp.maximum(c1, c2), c3)
    return slope_l_use, slope_u_use, l_bias, u_bias


def _sc_fill_body(lb_hbm, ub_hbm, sl_hbm, su_hbm, zrows_hbm,
                  out_hbm, in_pv, pv, buf, stage_sem, sem):
    c = lax.axis_index("c")
    s = lax.axis_index("s")
    w = s * 2 + c
    row0 = w * _RPW

    d0 = pltpu.async_copy(lb_hbm.at[pl.ds(row0, _RPW)], in_pv.at[0], stage_sem)
    d1 = pltpu.async_copy(ub_hbm.at[pl.ds(row0, _RPW)], in_pv.at[1], stage_sem)
    d2 = pltpu.async_copy(sl_hbm.at[pl.ds(row0, _RPW)], in_pv.at[2], stage_sem)
    d3 = pltpu.async_copy(su_hbm.at[pl.ds(row0, _RPW)], in_pv.at[3], stage_sem)
    zd = pltpu.async_copy(zrows_hbm, buf, sem)
    d0.wait()
    d1.wait()
    d2.wait()
    d3.wait()

    for g in range(_RPW // 16):
        o = g * 16
        slu16, suu16, lb16, ub16 = _params16(
            in_pv[0, pl.ds(o, 16)], in_pv[1, pl.ds(o, 16)],
            in_pv[2, pl.ds(o, 16)], in_pv[3, pl.ds(o, 16)])
        pv[0, pl.ds(o, 16)] = slu16
        pv[1, pl.ds(o, 16)] = suu16
        pv[2, pl.ds(o, 16)] = lb16
        pv[3, pl.ds(o, 16)] = ub16
    zd.wait()

    lane = lax.iota(jnp.int32, 16)
    rr = lane & 7
    mlow = lane < 8
    zero16i = jnp.zeros((16,), jnp.int32)
    one16i = jnp.full((16,), 1, jnp.int32)
    last16i = jnp.full((16,), _N, jnp.int32)
    zval = jnp.zeros((16,), jnp.float32)

    def chunk(i, carry):
        l0 = i * _K
        r0 = row0 + l0
        dcol = r0 + rr
        slu16 = pv[0, pl.ds(l0, 16)]
        suu16 = pv[1, pl.ds(l0, 16)]
        lb16 = pv[2, pl.ds(l0, 16)]
        ub16 = pv[3, pl.ds(l0, 16)]
        plsc.store_scatter(buf, [rr, zero16i, dcol], slu16, mask=mlow)
        plsc.store_scatter(buf, [rr, one16i, dcol], suu16, mask=mlow)
        plsc.store_scatter(buf, [rr, zero16i, last16i], lb16, mask=mlow)
        plsc.store_scatter(buf, [rr, one16i, last16i], ub16, mask=mlow)
        pltpu.async_copy(buf, out_hbm.at[pl.ds(r0, _K)], sem).wait()
        plsc.store_scatter(buf, [rr, zero16i, dcol], zval, mask=mlow)
        plsc.store_scatter(buf, [rr, one16i, dcol], zval, mask=mlow)
        plsc.store_scatter(buf, [rr, zero16i, last16i], zval, mask=mlow)
        plsc.store_scatter(buf, [rr, one16i, last16i], zval, mask=mlow)
        return carry

    lax.fori_loop(0, _NCHUNK, chunk, 0)

    @pl.when(w == 31)
    def _():
        m_idx = jnp.where(lane < 8, 0, 1).astype(jnp.int32)
        one_val = jnp.full((16,), 1.0, jnp.float32)
        plsc.store_scatter(buf, [zero16i, m_idx, last16i], one_val)
        pltpu.async_copy(buf.at[pl.ds(0, 1)],
                         out_hbm.at[pl.ds(_N, 1)], sem).wait()


def _sc_fill(lb, ub, slope_l, slope_u):
    zrows = jnp.zeros((_K, 2, _D), jnp.float32)
    mesh = plsc.VectorSubcoreMesh(core_axis_name="c", subcore_axis_name="s")
    fill = functools.partial(
        pl.kernel,
        mesh=mesh,
        compiler_params=pltpu.CompilerParams(needs_layout_passes=False),
        out_type=jax.ShapeDtypeStruct((_D, 2, _D), jnp.float32),
        scratch_types=[
            pltpu.VMEM((4, _RPW), jnp.float32),
            pltpu.VMEM((4, _PVW), jnp.float32),
            pltpu.VMEM((_K, 2, _D), jnp.float32),
            pltpu.SemaphoreType.DMA,
            pltpu.SemaphoreType.DMA,
        ],
    )(_sc_fill_body)
    return fill(lb, ub, slope_l, slope_u, zrows)


def kernel(lb, ub, slope_l, slope_u):
    out = _sc_fill(lb, ub, slope_l, slope_u)
    return jnp.transpose(out, (1, 0, 2))
